# Initial kernel scaffold; baseline (speedup 1.0000x reference)
import jax
import jax.numpy as jnp
from jax.experimental import pallas as pl


def kernel(x, table, W, b):
    raise NotImplementedError("write your pallas kernel here")


# speedup vs baseline: 14.1677x; 14.1677x over previous
import functools

import jax
import jax.numpy as jnp
from jax import lax
from jax.experimental import pallas as pl
from jax.experimental.pallas import tpu as pltpu
from jax.experimental.pallas import tpu_sc as plsc


def _proj_body(tbl_ref, w_ref, bias_ref, out_ref):
    r = lax.dot_general(
        w_ref[...], tbl_ref[...],
        (((1,), (1,)), ((), ())),
        preferred_element_type=jnp.float32,
    )
    out_ref[...] = r[0] + bias_ref[0, 0]


def _project_table(table, w_scaled, bias_scaled, blk=2048):
    V, D = table.shape
    grid = pl.cdiv(V, blk)
    return pl.pallas_call(
        _proj_body,
        grid=(grid,),
        in_specs=[
            pl.BlockSpec((blk, D), lambda i: (i, 0)),
            pl.BlockSpec((1, D), lambda i: (0, 0)),
            pl.BlockSpec(memory_space=pltpu.SMEM),
        ],
        out_specs=pl.BlockSpec((blk,), lambda i: (i,)),
        out_shape=jax.ShapeDtypeStruct((V,), jnp.float32),
    )(table, w_scaled, bias_scaled)


def _make_sc_gather(V, S, B, NC, NS, C):
    NW = NC * NS
    per_w = B // NW
    n_chunks = per_w // C
    n_seg = C // 16

    mesh = plsc.VectorSubcoreMesh(core_axis_name="c", subcore_axis_name="s")

    @functools.partial(
        pl.kernel,
        mesh=mesh,
        out_type=jax.ShapeDtypeStruct((B,), jnp.float32),
        scratch_types=[
            pltpu.VMEM((S * C,), jnp.int32),
            pltpu.VMEM((S * C,), jnp.float32),
            pltpu.VMEM((C,), jnp.float32),
            pltpu.SemaphoreType.DMA,
        ],
    )
    def sc_gather(v_hbm, xp_hbm, out_hbm, idx_v, vals_v, outb_v, sem):
        wid = lax.axis_index("s") * NC + lax.axis_index("c")

        def chunk_body(ci, carry):
            base = wid * per_w + ci * C
            pltpu.sync_copy(xp_hbm.at[pl.ds(base * S, S * C)], idx_v)
            pltpu.async_copy(v_hbm.at[idx_v], vals_v, sem).wait()

            def s_body(si, acc):
                return tuple(
                    acc[jj] + vals_v[pl.ds(si * C + jj * 16, 16)]
                    for jj in range(n_seg)
                )

            acc0 = tuple(jnp.zeros((16,), jnp.float32) for _ in range(n_seg))
            acc = lax.fori_loop(0, S, s_body, acc0)

            for jj in range(n_seg):
                z = acc[jj]
                outb_v[pl.ds(jj * 16, 16)] = 1.0 / (1.0 + jnp.exp(-z))
            pltpu.sync_copy(outb_v, out_hbm.at[pl.ds(base, C)])
            return carry

        lax.fori_loop(0, n_chunks, chunk_body, 0)

    return sc_gather


def kernel(x, table, W, b):
    B, S = x.shape
    V, D = table.shape

    x = x.astype(jnp.int32)

    info = plsc.get_sparse_core_info()
    NC, NS = info.num_cores, info.num_subcores
    NW = NC * NS
    C = 128
    n_chunks = B // (NW * C)
    xp = x.reshape(NW, n_chunks, C, S).swapaxes(2, 3).reshape(B * S)

    w_scaled = (W * (1.0 / S)).astype(jnp.float32)
    bias_scaled = (b * (1.0 / S)).reshape(1, 1).astype(jnp.float32)

    v = _project_table(table, w_scaled, bias_scaled)

    sc_gather = _make_sc_gather(V, S, B, NC, NS, C)
    return sc_gather(v, xp)


# speedup vs baseline: 54.9429x; 3.8780x over previous
import functools

import jax
import jax.numpy as jnp
from jax import lax
from jax.experimental import pallas as pl
from jax.experimental.pallas import tpu as pltpu
from jax.experimental.pallas import tpu_sc as plsc


def _proj_body(tblT_ref, w_ref, bias_ref, out_ref):
    r = lax.dot_general(
        w_ref[...], tblT_ref[...],
        (((1,), (0,)), ((), ())),
        preferred_element_type=jnp.float32,
    )
    out_ref[...] = r[0] + bias_ref[0, 0]


def _project_table(tableT, w_scaled, bias_scaled, blk=65536):
    D, V = tableT.shape
    grid = pl.cdiv(V, blk)
    return pl.pallas_call(
        _proj_body,
        grid=(grid,),
        in_specs=[
            pl.BlockSpec((D, blk), lambda i: (0, i)),
            pl.BlockSpec((1, D), lambda i: (0, 0)),
            pl.BlockSpec(memory_space=pltpu.SMEM),
        ],
        out_specs=pl.BlockSpec((blk,), lambda i: (i,)),
        out_shape=jax.ShapeDtypeStruct((V,), jnp.float32),
    )(tableT, w_scaled, bias_scaled)


def _make_sc_gather(V, S, B, NC, NS, C):
    NW = NC * NS
    per_w = B // NW
    n_chunks = per_w // C
    n_seg = C // 16

    mesh = plsc.VectorSubcoreMesh(core_axis_name="c", subcore_axis_name="s")

    @functools.partial(
        pl.kernel,
        mesh=mesh,
        out_type=jax.ShapeDtypeStruct((B,), jnp.float32),
        scratch_types=[
            pltpu.VMEM((S * C,), jnp.int32),
            pltpu.VMEM((S * C,), jnp.float32),
            pltpu.VMEM((C,), jnp.float32),
            pltpu.SemaphoreType.DMA,
        ],
    )
    def sc_gather(v_hbm, xp_hbm, out_hbm, idx_v, vals_v, outb_v, sem):
        wid = lax.axis_index("s") * NC + lax.axis_index("c")

        def chunk_body(ci, carry):
            base = wid * per_w + ci * C
            pltpu.sync_copy(xp_hbm.at[pl.ds(base * S, S * C)], idx_v)
            pltpu.async_copy(v_hbm.at[idx_v], vals_v, sem).wait()

            def s_body(si, acc):
                return tuple(
                    acc[jj] + vals_v[pl.ds(si * C + jj * 16, 16)]
                    for jj in range(n_seg)
                )

            acc0 = tuple(jnp.zeros((16,), jnp.float32) for _ in range(n_seg))
            acc = lax.fori_loop(0, S, s_body, acc0)

            for jj in range(n_seg):
                z = acc[jj]
                outb_v[pl.ds(jj * 16, 16)] = 1.0 / (1.0 + jnp.exp(-z))
            pltpu.sync_copy(outb_v, out_hbm.at[pl.ds(base, C)])
            return carry

        lax.fori_loop(0, n_chunks, chunk_body, 0)

    return sc_gather


def kernel(x, table, W, b):
    B, S = x.shape
    V, D = table.shape

    x = x.astype(jnp.int32)

    info = plsc.get_sparse_core_info()
    NC, NS = info.num_cores, info.num_subcores
    NW = NC * NS
    C = 128
    n_chunks = B // (NW * C)
    xp = x.reshape(NW, n_chunks, C, S).swapaxes(2, 3).reshape(B * S)

    w_scaled = (W * (1.0 / S)).astype(jnp.float32)
    bias_scaled = (b * (1.0 / S)).reshape(1, 1).astype(jnp.float32)

    tT = jnp.swapaxes(table, 0, 1)
    v = _project_table(tT, w_scaled, bias_scaled)

    sc_gather = _make_sc_gather(V, S, B, NC, NS, C)
    return sc_gather(v, xp)
